# baseline (speedup 1.0000x reference)
import functools

import jax
import jax.numpy as jnp
from jax import lax
from jax.experimental import pallas as pl
from jax.experimental.pallas import tpu as pltpu
from jax.experimental.pallas import tpu_sc as plsc

BATCH = 4096
EMBED_DIM = 64
LANES = 16

_info = plsc.get_sparse_core_info()
NUM_CORES = _info.num_cores
NUM_SUBCORES = _info.num_subcores
NUM_WORKERS = NUM_CORES * NUM_SUBCORES
B_PER_W = BATCH // NUM_WORKERS
GROUPS = B_PER_W // LANES


@functools.partial(
    pl.kernel,
    mesh=plsc.VectorSubcoreMesh(core_axis_name="c", subcore_axis_name="s"),
    compiler_params=pltpu.CompilerParams(
        needs_layout_passes=False, use_tc_tiling_on_sc=False),
    out_type=jax.ShapeDtypeStruct((BATCH,), jnp.float32),
    scratch_types=[
        pltpu.VMEM((B_PER_W,), jnp.int32),
        pltpu.VMEM((B_PER_W,), jnp.int32),
        pltpu.VMEM((B_PER_W, EMBED_DIM), jnp.float32),
        pltpu.VMEM((B_PER_W, EMBED_DIM), jnp.float32),
        pltpu.VMEM((B_PER_W,), jnp.float32),
        pltpu.SemaphoreType.DMA,
        pltpu.SemaphoreType.DMA,
    ],
)
def _mf_dot_kernel(uidx_hbm, iidx_hbm, utab_hbm, itab_hbm, out_hbm,
                   uidx_v, iidx_v, urows_v, irows_v, out_v, usem, isem):
    wid = lax.axis_index("s") * NUM_CORES + lax.axis_index("c")
    base = wid * B_PER_W

    pltpu.sync_copy(uidx_hbm.at[pl.ds(base, B_PER_W)], uidx_v)
    pltpu.sync_copy(iidx_hbm.at[pl.ds(base, B_PER_W)], iidx_v)

    ucp = pltpu.async_copy(utab_hbm.at[uidx_v], urows_v, usem)
    icp = pltpu.async_copy(itab_hbm.at[iidx_v], irows_v, isem)
    ucp.wait()
    icp.wait()

    lane = lax.iota(jnp.int32, LANES)

    def group_body(g, _):
        res = jnp.zeros((LANES,), jnp.float32)
        for r in range(LANES):
            b = g * LANES + r
            acc = jnp.zeros((LANES,), jnp.float32)
            for c in range(EMBED_DIM // LANES):
                uv = urows_v[b, pl.ds(c * LANES, LANES)]
                iv = irows_v[b, pl.ds(c * LANES, LANES)]
                acc = acc + uv * iv
            res = jnp.where(lane == r, jnp.sum(acc), res)
        out_v[pl.ds(g * LANES, LANES)] = res
        return _

    lax.fori_loop(0, GROUPS, group_body, 0)

    pltpu.sync_copy(out_v, out_hbm.at[pl.ds(base, B_PER_W)])


def kernel(user_indices, item_indices, user_table, item_table):
    uidx = jnp.asarray(user_indices, jnp.int32)
    iidx = jnp.asarray(item_indices, jnp.int32)
    return _mf_dot_kernel(uidx, iidx, user_table, item_table)


# speedup vs baseline: 1.4677x; 1.4677x over previous
import functools

import jax
import jax.numpy as jnp
from jax import lax
from jax.experimental import pallas as pl
from jax.experimental.pallas import tpu as pltpu
from jax.experimental.pallas import tpu_sc as plsc

BATCH = 4096
EMBED_DIM = 64
LANES = 16

_info = plsc.get_sparse_core_info()
NUM_CORES = _info.num_cores
NUM_SUBCORES = _info.num_subcores
NUM_WORKERS = NUM_CORES * NUM_SUBCORES
B_PER_W = BATCH // NUM_WORKERS
GROUPS = B_PER_W // LANES


@functools.partial(
    pl.kernel,
    mesh=plsc.VectorSubcoreMesh(core_axis_name="c", subcore_axis_name="s"),
    compiler_params=pltpu.CompilerParams(needs_layout_passes=False),
    out_type=jax.ShapeDtypeStruct((BATCH,), jnp.float32),
    scratch_types=[
        pltpu.VMEM((B_PER_W,), jnp.int32),
        pltpu.VMEM((B_PER_W,), jnp.int32),
        pltpu.VMEM((B_PER_W, EMBED_DIM), jnp.float32),
        pltpu.VMEM((B_PER_W, EMBED_DIM), jnp.float32),
        pltpu.VMEM((B_PER_W,), jnp.float32),
        pltpu.SemaphoreType.DMA,
        pltpu.SemaphoreType.DMA,
    ],
)
def _mf_dot_kernel(uidx_hbm, iidx_hbm, utab_hbm, itab_hbm, out_hbm,
                   uidx_v, iidx_v, urows_v, irows_v, out_v, usem, isem):
    wid = lax.axis_index("s") * NUM_CORES + lax.axis_index("c")
    base = wid * B_PER_W

    pltpu.sync_copy(uidx_hbm.at[pl.ds(base, B_PER_W)], uidx_v)
    pltpu.sync_copy(iidx_hbm.at[pl.ds(base, B_PER_W)], iidx_v)

    def fire_body(g, _):
        uvec = uidx_v[pl.ds(g * LANES, LANES)]
        ivec = iidx_v[pl.ds(g * LANES, LANES)]
        for r in range(LANES):
            b = g * LANES + r
            pltpu.async_copy(utab_hbm.at[uvec[r]], urows_v.at[b], usem)
            pltpu.async_copy(itab_hbm.at[ivec[r]], irows_v.at[b], isem)
        return _

    lax.fori_loop(0, GROUPS, fire_body, 0)
    pltpu.make_async_copy(utab_hbm.at[pl.ds(0, B_PER_W)], urows_v, usem).wait()
    pltpu.make_async_copy(itab_hbm.at[pl.ds(0, B_PER_W)], irows_v, isem).wait()

    lane = lax.iota(jnp.int32, LANES)

    def group_body(g, _):
        res = jnp.zeros((LANES,), jnp.float32)
        for r in range(LANES):
            b = g * LANES + r
            acc = jnp.zeros((LANES,), jnp.float32)
            for c in range(EMBED_DIM // LANES):
                uv = urows_v[b, pl.ds(c * LANES, LANES)]
                iv = irows_v[b, pl.ds(c * LANES, LANES)]
                acc = acc + uv * iv
            res = jnp.where(lane == r, jnp.sum(acc), res)
        out_v[pl.ds(g * LANES, LANES)] = res
        return _

    lax.fori_loop(0, GROUPS, group_body, 0)

    pltpu.sync_copy(out_v, out_hbm.at[pl.ds(base, B_PER_W)])


def kernel(user_indices, item_indices, user_table, item_table):
    uidx = jnp.asarray(user_indices, jnp.int32)
    iidx = jnp.asarray(item_indices, jnp.int32)
    return _mf_dot_kernel(uidx, iidx, user_table, item_table)
